# Initial kernel scaffold; baseline (speedup 1.0000x reference)
import jax
import jax.numpy as jnp
from jax.experimental import pallas as pl


def kernel(x):
    raise NotImplementedError("write your pallas kernel here")


# speedup vs baseline: 1.4307x; 1.4307x over previous
import jax
import jax.numpy as jnp
from jax.experimental import pallas as pl

_QMAX = 127.0
_COL_BLOCK = 512


def _fq_kernel(x_ref, o_ref):
    v = x_ref[0, 0]
    amax = jnp.max(jnp.abs(v), axis=0, keepdims=True)
    scale = amax * (1.0 / _QMAX)
    scale = jnp.where(scale == 0.0, 1.0, scale)
    q = jnp.clip(jnp.round(v / scale), -(_QMAX + 1.0), _QMAX)
    o_ref[0, 0] = q * scale


def kernel(x):
    BS, H, N, M = x.shape
    grid = (BS * H, M // _COL_BLOCK)
    spec = pl.BlockSpec((1, 1, N, _COL_BLOCK), lambda h, j: (0, h, 0, j))
    return pl.pallas_call(
        _fq_kernel,
        grid=grid,
        in_specs=[spec],
        out_specs=spec,
        out_shape=jax.ShapeDtypeStruct(x.shape, x.dtype),
    )(x)


# speedup vs baseline: 1.4638x; 1.0231x over previous
import jax
import jax.numpy as jnp
from jax.experimental import pallas as pl

_QMAX = 127.0
_COL_BLOCK = 1024


def _fq_kernel(x_ref, o_ref):
    v = x_ref[0, 0]
    amax = jnp.max(jnp.abs(v), axis=0, keepdims=True)
    scale = amax * (1.0 / _QMAX)
    scale = jnp.where(scale == 0.0, 1.0, scale)
    q = jnp.clip(jnp.round(v / scale), -(_QMAX + 1.0), _QMAX)
    o_ref[0, 0] = q * scale


def kernel(x):
    BS, H, N, M = x.shape
    grid = (BS * H, M // _COL_BLOCK)
    spec = pl.BlockSpec((1, 1, N, _COL_BLOCK), lambda h, j: (0, h, 0, j))
    return pl.pallas_call(
        _fq_kernel,
        grid=grid,
        in_specs=[spec],
        out_specs=spec,
        out_shape=jax.ShapeDtypeStruct(x.shape, x.dtype),
    )(x)
